# Initial kernel scaffold; baseline (speedup 1.0000x reference)
import jax
import jax.numpy as jnp
from jax.experimental import pallas as pl


def kernel(x, edge_index, edge_weight, batch, num_graphs, wr0, wn0, b0, wr1, wn1, b1, wr2, wn2, b2, wr3, wn3, b3, wr4, wn4, b4, wr5, wn5, b5, fc1_w, fc1_b, fc2_w, fc2_b):
    raise NotImplementedError("write your pallas kernel here")


# speedup vs baseline: 4.3771x; 4.3771x over previous
import functools

import jax
import jax.numpy as jnp
from jax import lax
from jax.experimental import pallas as pl
from jax.experimental.pallas import tpu as pltpu
from jax.experimental.pallas import tpu_sc as plsc

N = 92160
E = 1474560
NG = 256
NPG = 360
CH = 128
NSUB = 16
NODES_PER_SUB = N // NSUB
GRAPHS_PER_SUB = NODES_PER_SUB // NPG
EDGES_PER_SUB = E // NSUB
CHUNKS_PER_SUB = EDGES_PER_SUB // CH
MM_TILE = 2048


def _mm_body(hlo_ref, hhi_ref, wr_ref, wn_ref, b_ref,
             blo_ref, bhi_ref, nlo_ref, nhi_ref):
    h = jnp.concatenate([hlo_ref[...], hhi_ref[...]], axis=1)
    base = jnp.dot(h, wr_ref[...], preferred_element_type=jnp.float32) + b_ref[...]
    hn = jnp.dot(h, wn_ref[...], preferred_element_type=jnp.float32)
    blo_ref[...] = base[:, :16]
    bhi_ref[...] = base[:, 16:]
    nlo_ref[...] = hn[:, :16]
    nhi_ref[...] = hn[:, 16:]


def _mm(h_lo, h_hi, wr32, wn32, b32):
    half = jax.ShapeDtypeStruct((N, 16), jnp.float32)
    grid = N // MM_TILE
    hspec = pl.BlockSpec((MM_TILE, 16), lambda i: (i, 0))
    wspec = pl.BlockSpec((32, 32), lambda i: (0, 0))
    bspec = pl.BlockSpec((1, 32), lambda i: (0, 0))
    return pl.pallas_call(
        _mm_body,
        grid=(grid,),
        in_specs=[hspec, hspec, wspec, wspec, bspec],
        out_specs=[hspec, hspec, hspec, hspec],
        out_shape=[half, half, half, half],
    )(h_lo, h_hi, wr32, wn32, b32)


def _sc_layer_fn(residual):
    mesh = plsc.VectorSubcoreMesh(core_axis_name="c", subcore_axis_name="s")
    half = jax.ShapeDtypeStruct((N, 16), jnp.float32)
    pooled = jax.ShapeDtypeStruct((NG, 16), jnp.float32)

    def body(src_hbm, dst_hbm, ew_hbm, hn_lo, hn_hi, base_lo, base_hi,
             h_lo, h_hi, hnext_lo, hnext_hi, pooled_lo, pooled_hi,
             agg_sh, idx_v, dstidx_v, ew_v, rows_v, node_v, res_v, pool_v, sem):
        cid = lax.axis_index("c")
        sid = lax.axis_index("s")

        def run_half(hn_h, base_h, h_h, hnext_h, pooled_h):
            node0 = sid * NODES_PER_SUB
            pltpu.sync_copy(base_h.at[pl.ds(node0, NODES_PER_SUB)],
                            agg_sh.at[pl.ds(node0, NODES_PER_SUB)])
            plsc.subcore_barrier()

            ebase = sid * EDGES_PER_SUB

            def chunk_body(ci, carry):
                e0 = ebase + ci * CH
                pltpu.sync_copy(src_hbm.at[pl.ds(e0, CH)], idx_v)
                pltpu.sync_copy(dst_hbm.at[pl.ds(e0, CH)], dstidx_v)
                pltpu.sync_copy(ew_hbm.at[pl.ds(e0, CH)], ew_v)
                pltpu.async_copy(hn_h.at[idx_v], rows_v, sem).wait()

                def scale_body(g, c2):
                    ew16 = ew_v[pl.ds(g * 16, 16)]
                    for j in range(16):
                        e = g * 16 + j
                        rows_v[e, :] = rows_v[e, :] * ew16[j]
                    return c2
                lax.fori_loop(0, CH // 16, scale_body, 0)
                pltpu.sync_copy(rows_v, agg_sh.at[dstidx_v], add=True)
                return carry

            lax.fori_loop(0, CHUNKS_PER_SUB, chunk_body, 0)
            plsc.subcore_barrier()

            def graph_body(g, carry):
                n0 = node0 + g * NPG
                pltpu.sync_copy(agg_sh.at[pl.ds(n0, NPG)], node_v)
                if residual:
                    pltpu.sync_copy(h_h.at[pl.ds(n0, NPG)], res_v)

                def row_body(r, mx):
                    v = jnp.maximum(node_v[r, :], 0.0)
                    if residual:
                        v = v + res_v[r, :]
                    node_v[r, :] = v
                    return jnp.maximum(mx, v)

                mx = lax.fori_loop(0, NPG, row_body,
                                   jnp.zeros((16,), jnp.float32), unroll=4)
                pool_v[g, :] = mx
                pltpu.sync_copy(node_v, hnext_h.at[pl.ds(n0, NPG)])
                return carry

            lax.fori_loop(0, GRAPHS_PER_SUB, graph_body, 0)
            pltpu.sync_copy(pool_v, pooled_h.at[pl.ds(sid * GRAPHS_PER_SUB,
                                                      GRAPHS_PER_SUB)])

        @pl.when(cid == 0)
        def _():
            run_half(hn_lo, base_lo, h_lo, hnext_lo, pooled_lo)

        @pl.when(cid == 1)
        def _():
            run_half(hn_hi, base_hi, h_hi, hnext_hi, pooled_hi)

    return pl.kernel(
        body,
        mesh=mesh,
        compiler_params=pltpu.CompilerParams(use_tc_tiling_on_sc=False),
        out_type=[half, half, pooled, pooled],
        scratch_types=[
            pltpu.VMEM_SHARED((N, 16), jnp.float32),
            pltpu.VMEM((CH,), jnp.int32),
            pltpu.VMEM((CH,), jnp.int32),
            pltpu.VMEM((CH,), jnp.float32),
            pltpu.VMEM((CH, 16), jnp.float32),
            pltpu.VMEM((NPG, 16), jnp.float32),
            pltpu.VMEM((NPG, 16), jnp.float32),
            pltpu.VMEM((GRAPHS_PER_SUB, 16), jnp.float32),
            pltpu.SemaphoreType.DMA,
        ],
    )


def _fc_body(*refs):
    pooled_refs = refs[:12]
    fc1w_ref, fc1b_ref, fc2w_ref, fc2b_ref, out_ref = refs[12:]
    parts = []
    for i in range(6):
        parts.append(pooled_refs[2 * i][...])
        parts.append(pooled_refs[2 * i + 1][...][:, :14])
    g = jnp.concatenate(parts, axis=1)
    z = jnp.maximum(
        jnp.dot(g, fc1w_ref[...], preferred_element_type=jnp.float32)
        + fc1b_ref[...], 0.0)
    out_ref[...] = (jnp.dot(z, fc2w_ref[...], preferred_element_type=jnp.float32)
                    + fc2b_ref[...])


def _fc(pooled_halves, fc1_w, fc1_b, fc2_w, fc2_b):
    return pl.pallas_call(
        _fc_body,
        out_shape=jax.ShapeDtypeStruct((NG, 2), jnp.float32),
    )(*pooled_halves, fc1_w, fc1_b.reshape(1, 50), fc2_w, fc2_b.reshape(1, 2))


def _pad_w(w):
    fi, fo = w.shape
    return jnp.zeros((32, 32), jnp.float32).at[:fi, :fo].set(w)


def kernel(x, edge_index, edge_weight, batch, num_graphs,
           wr0, wn0, b0, wr1, wn1, b1, wr2, wn2, b2,
           wr3, wn3, b3, wr4, wn4, b4, wr5, wn5, b5,
           fc1_w, fc1_b, fc2_w, fc2_b):
    src = edge_index[0]
    dst = edge_index[1]
    params = [(wr0, wn0, b0), (wr1, wn1, b1), (wr2, wn2, b2),
              (wr3, wn3, b3), (wr4, wn4, b4), (wr5, wn5, b5)]

    h_lo = jnp.pad(x, ((0, 0), (0, 16 - x.shape[1])))
    h_hi = jnp.zeros((N, 16), jnp.float32)

    sc_first = _sc_layer_fn(residual=False)
    sc_rest = _sc_layer_fn(residual=True)

    pooled_halves = []
    for i, (wr, wn, b) in enumerate(params):
        wr32 = _pad_w(wr)
        wn32 = _pad_w(wn)
        b32 = jnp.zeros((1, 32), jnp.float32).at[0, :b.shape[0]].set(b)
        base_lo, base_hi, hn_lo, hn_hi = _mm(h_lo, h_hi, wr32, wn32, b32)
        sc = sc_first if i == 0 else sc_rest
        h_lo, h_hi, p_lo, p_hi = sc(src, dst, edge_weight,
                                    hn_lo, hn_hi, base_lo, base_hi,
                                    h_lo, h_hi)
        pooled_halves.append(p_lo)
        pooled_halves.append(p_hi)

    fc = _fc(pooled_halves, fc1_w, fc1_b, fc2_w, fc2_b)
    reg = jnp.zeros((1, 1), jnp.float32)
    return (fc, reg)
